# Initial kernel scaffold; baseline (speedup 1.0000x reference)
import jax
import jax.numpy as jnp
from jax.experimental import pallas as pl


def kernel(prev_query_ids, curr_query_ids, query_positions, prev_updated_pos, prev_updated_occlusion, prev_updated_certainty, prev_updated_velocity, prev_mconf, prev_pips_mem):
    raise NotImplementedError("write your pallas kernel here")


# speedup vs baseline: 4.1613x; 4.1613x over previous
import jax
import jax.numpy as jnp
from jax import lax
from jax.experimental import pallas as pl
from jax.experimental.pallas import tpu as pltpu
from jax.experimental.pallas import tpu_sc as plsc

N_PREV = 16384
N_ACT = 16384
ROW_W = 8 * 128
NC, NS, L = 2, 16, 16
NW = NC * NS
RPW = N_ACT // NW
VPW = RPW // L
ZROWS = 32
NZ = RPW // ZROWS


def _sc_body(prev_ids, curr_ids, packed_prev, packed_def, zeros_blk, pips_in,
             small_out, pips_out,
             prev_v, curr_v, idx_v, found_v, def_v, zbuf_v, row_v,
             sem_in, sem_z, sem_s, sem_row):
    cid = lax.axis_index("c")
    sid = lax.axis_index("s")
    wid = sid * NC + cid
    base = wid * RPW

    cp_prev = pltpu.async_copy(prev_ids, prev_v, sem_in)
    cp_curr = pltpu.async_copy(curr_ids.at[pl.ds(base, RPW)], curr_v, sem_in)
    cp_def = pltpu.async_copy(packed_def.at[pl.ds(base, RPW), :], def_v,
                              sem_in)
    pltpu.async_copy(zeros_blk, zbuf_v, sem_z).wait()

    zcopies = []
    for z in range(NZ):
        zcopies.append(pltpu.async_copy(
            zbuf_v,
            pips_out.at[pl.ds((base + z * ZROWS) * ROW_W, ZROWS * ROW_W)],
            sem_z))

    cp_prev.wait()
    cp_curr.wait()

    def _bs(v, c):
        q = curr_v[pl.ds(v * L, L)]
        b = jnp.zeros((L,), jnp.int32)
        step = N_PREV // 2
        while step >= 1:
            probe = plsc.load_gather(prev_v, [b + (step - 1)])
            b = jnp.where(probe < q, b + step, b)
            step //= 2
        last = plsc.load_gather(prev_v, [b])
        pos = jnp.minimum(b + jnp.where(last < q, 1, 0), N_PREV - 1)
        val = plsc.load_gather(prev_v, [pos])
        idx_v[pl.ds(v * L, L)] = pos
        found_v[pl.ds(v * L, L)] = (val == q).astype(jnp.int32)
        return c

    lax.fori_loop(0, VPW, _bs, 0)

    cp_def.wait()
    for cp in zcopies:
        cp.wait()

    def _patch(r, c):
        f = found_v[pl.ds(r, L)][0]

        @pl.when(f != 0)
        def _():
            src = idx_v[pl.ds(r, L)][0]
            cp_row = pltpu.async_copy(
                pips_in.at[pl.ds(src * ROW_W, ROW_W)], row_v, sem_row)
            cp_sml = pltpu.async_copy(
                packed_prev.at[pl.ds(src, 1), :], def_v.at[pl.ds(r, 1), :],
                sem_row)
            cp_row.wait()
            cp_sml.wait()
            pltpu.async_copy(
                row_v, pips_out.at[pl.ds((base + r) * ROW_W, ROW_W)],
                sem_row).wait()
        return c

    lax.fori_loop(0, RPW, _patch, 0)
    pltpu.async_copy(def_v, small_out.at[pl.ds(base, RPW), :], sem_s).wait()


def kernel(prev_query_ids, curr_query_ids, query_positions, prev_updated_pos,
           prev_updated_occlusion, prev_updated_certainty,
           prev_updated_velocity, prev_mconf, prev_pips_mem):
    f32 = jnp.float32
    packed_prev = jnp.concatenate([
        prev_updated_pos[0], prev_updated_occlusion[0],
        prev_updated_certainty[0], prev_updated_velocity[0], prev_mconf[0],
        jnp.zeros((N_PREV, 1), f32)], axis=1)
    packed_def = jnp.concatenate([
        query_positions[0],
        jnp.zeros((N_ACT, 1), f32),
        jnp.full((N_ACT, 1), 100.0, f32),
        jnp.zeros((N_ACT, 2), f32),
        jnp.full((N_ACT, 1), 10.0, f32),
        jnp.zeros((N_ACT, 1), f32)], axis=1)
    zeros_blk = jnp.zeros((ZROWS * ROW_W,), f32)
    pips_flat = prev_pips_mem.reshape(-1)

    mesh = plsc.VectorSubcoreMesh(core_axis_name="c", subcore_axis_name="s")
    small, pips = pl.kernel(
        _sc_body,
        out_type=[
            jax.ShapeDtypeStruct((N_ACT, 8), f32),
            jax.ShapeDtypeStruct((N_ACT * ROW_W,), f32),
        ],
        mesh=mesh,
        compiler_params=pltpu.CompilerParams(needs_layout_passes=False),
        scratch_types=[
            pltpu.VMEM((N_PREV,), jnp.int32),
            pltpu.VMEM((RPW,), jnp.int32),
            pltpu.VMEM((RPW + L,), jnp.int32),
            pltpu.VMEM((RPW + L,), jnp.int32),
            pltpu.VMEM((RPW, 8), f32),
            pltpu.VMEM((ZROWS * ROW_W,), f32),
            pltpu.VMEM((ROW_W,), f32),
            pltpu.SemaphoreType.DMA,
            pltpu.SemaphoreType.DMA,
            pltpu.SemaphoreType.DMA,
            pltpu.SemaphoreType.DMA,
        ],
    )(prev_query_ids, curr_query_ids, packed_prev, packed_def, zeros_blk,
      pips_flat)

    updated_pos = small[:, 0:2][None]
    updated_occlusion = small[:, 2:3][None]
    updated_certainty = small[:, 3:4][None]
    updated_velocity = small[:, 4:6][None]
    mconf_logits_coarse = small[:, 6:7][None]
    new_pips_mem = pips.reshape(N_ACT, 8, 128)
    return (updated_pos, updated_occlusion, updated_certainty,
            updated_velocity, mconf_logits_coarse, new_pips_mem)
